# baseline (speedup 1.0000x reference)
import functools

import jax
import jax.numpy as jnp
from jax import lax
from jax.experimental import pallas as pl
from jax.experimental.pallas import tpu as pltpu
from jax.experimental.pallas import tpu_sc as plsc

_RUN = 1600
_RPC = 16
_CHUNK = _RPC * _RUN
_NCHUNKS = 16


def _sc_body(x_hbm, out_hbm, buf, res, sem0, sem1, *, scale):
    c_ax = lax.axis_index("c")
    s_ax = lax.axis_index("s")
    wid = c_ax * 16 + s_ax
    n = wid // 2
    h = wid % 2

    sems = (sem0, sem1)

    def start(q, b):
        m, g = q % 2, q // 2
        row = n * 512 + m * 256 + h * 128 + g * 16
        return pltpu.async_copy(
            x_hbm.at[pl.ds(row, _RPC), :], buf.at[b], sems[b])

    lane_iota = lax.iota(jnp.int32, 16)
    lane_zero = jnp.zeros((16,), jnp.int32)

    copies = [None, None]
    copies[0] = start(0, 0)
    for q in range(_NCHUNKS):
        b = q % 2
        if q + 1 < _NCHUNKS:
            copies[(q + 1) % 2] = start(q + 1, (q + 1) % 2)
        copies[b].wait()
        m, g = q % 2, q // 2
        bufq = buf.at[b]

        def gbody(k0, acc, _bufq=bufq):
            col = k0 * 16
            for dk in range(16):
                acc = acc + plsc.load_gather(
                    _bufq, [lane_iota, lane_zero + (col + dk)])
            return acc

        acc = lax.fori_loop(0, _RUN // 16, gbody, jnp.zeros((16,), jnp.float32))
        sl = pl.ds(g * 16, 16)
        if m == 0:
            res[sl] = acc
        else:
            res[sl] = (res[sl] + acc) * scale

    pltpu.sync_copy(res, out_hbm.at[pl.ds(n * 256 + h * 128, 128)])


def kernel(x, protos):
    N, M, C, T, V = x.shape
    P = protos.shape[0]
    scale = 1.0 / (M * P)
    xf = x.reshape(N * M * C, T * V)

    mesh = plsc.VectorSubcoreMesh(core_axis_name="c", subcore_axis_name="s")
    out = pl.kernel(
        functools.partial(_sc_body, scale=scale),
        mesh=mesh,
        compiler_params=pltpu.CompilerParams(
            use_tc_tiling_on_sc=False, needs_layout_passes=False),
        out_type=jax.ShapeDtypeStruct((N * C,), jnp.float32),
        scratch_types=[
            pltpu.VMEM((2, _RPC, _RUN), jnp.float32),
            pltpu.VMEM((128,), jnp.float32),
            pltpu.SemaphoreType.DMA,
            pltpu.SemaphoreType.DMA,
        ],
    )(xf)
    return out.reshape(N, C)


# speedup vs baseline: 5.2096x; 5.2096x over previous
import functools

import jax
import jax.numpy as jnp
from jax.experimental import pallas as pl

_CH = 400


def _reduce_body(x_ref, o_ref, *, scale):
    s = pl.program_id(1)
    partial = jnp.sum(x_ref[...], axis=1)[:, None, :] * scale

    @pl.when(s == 0)
    def _init():
        o_ref[...] = partial

    @pl.when(s != 0)
    def _acc():
        o_ref[...] += partial


def kernel(x, protos):
    N, M, C, T, V = x.shape
    P = protos.shape[0]
    scale = 1.0 / (M * P)

    xt = jnp.transpose(x, (0, 1, 4, 3, 2)).reshape(N, M * V * T, C)
    rows = M * V * T
    steps = rows // _CH

    out = pl.pallas_call(
        functools.partial(_reduce_body, scale=scale),
        grid=(N, steps),
        in_specs=[pl.BlockSpec((1, _CH, C), lambda n, s: (n, s, 0))],
        out_specs=pl.BlockSpec((1, 1, C), lambda n, s: (n, 0, 0)),
        out_shape=jax.ShapeDtypeStruct((N, 1, C), x.dtype),
    )(xt)
    return out.reshape(N, C)
